# baseline (device time: 58058 ns/iter reference)
import functools
import os

import jax
import jax.numpy as jnp
from jax import lax
from jax.experimental import pallas as pl
from jax.experimental.pallas import tpu as pltpu

N_DEV = 4

_SKIP_RING = bool(int(os.environ.get("KERNEL_SKIP_RING", "0")))
_SKIP_COMPUTE = bool(int(os.environ.get("KERNEL_SKIP_COMPUTE", "0")))
_RING_MODE = os.environ.get("KERNEL_RING_MODE", "full")
_COMPUTE_MODE = int(os.environ.get("KERNEL_COMPUTE_MODE", "0"))


def kernel(Q, K, V):
    b, s_loc, h, d = Q.shape
    h2 = h // 2
    d2 = 2 * d
    scale = d ** -0.5

    def prep(x):
        return x.reshape(b, s_loc, h2, d2).transpose(0, 2, 1, 3)

    Qp = prep((Q * scale).astype(jnp.bfloat16))
    Kp = prep(K.astype(jnp.bfloat16))
    Vp = prep(V.astype(jnp.bfloat16))

    def body(q_ref, k_ref, v_ref, out_ref,
             kbuf, vbuf, accbuf, lbuf, send_sems, recv_sems):
        my = lax.axis_index("i")
        left = (my - 1) % N_DEV
        right = (my + 1) % N_DEV

        barrier = pltpu.get_barrier_semaphore()
        for nbr in (left, right):
            pl.semaphore_signal(
                barrier, inc=1,
                device_id=(nbr,), device_id_type=pl.DeviceIdType.MESH,
            )
        pl.semaphore_wait(barrier, 2)

        def remote(src, dst, sem_idx, target):
            return pltpu.make_async_remote_copy(
                src_ref=src, dst_ref=dst,
                send_sem=send_sems.at[sem_idx],
                recv_sem=recv_sems.at[sem_idx],
                device_id=(target,),
                device_id_type=pl.DeviceIdType.MESH,
            )

        H = [slice(0, h2 // 2), slice(h2 // 2, h2)]
        k_r = [remote(k_ref.at[:, H[i]], kbuf.at[0, :, H[i]], 0 + i, right)
               for i in range(2)]
        v_r = [remote(v_ref.at[:, H[i]], vbuf.at[0, :, H[i]], 2 + i, right)
               for i in range(2)]
        k_l = [remote(k_ref.at[:, H[i]], kbuf.at[1, :, H[i]], 4 + i, left)
               for i in range(2)]
        v_l = [remote(v_ref.at[:, H[i]], vbuf.at[1, :, H[i]], 6 + i, left)
               for i in range(2)]
        k_fwd = [remote(kbuf.at[0, :, H[i]], kbuf.at[2, :, H[i]], 8 + i, right)
                 for i in range(2)]
        v_fwd = [remote(vbuf.at[1, :, H[i]], vbuf.at[2, :, H[i]], 10 + i, left)
                 for i in range(2)]

        _do_v = _RING_MODE in ("full", "hop0")
        _do_fwd = _RING_MODE == "full"
        if not _SKIP_RING:
            for i in range(2):
                k_r[i].start()
                if _do_v:
                    v_r[i].start()
            for i in range(2):
                k_l[i].start()
                if _do_v:
                    v_l[i].start()

        def head_chunk(q_t, k_t, v_t):
            st = lax.dot_general(
                q_t, k_t,
                (((1,), (1,)), ((), ())),
                preferred_element_type=jnp.float32,
            )
            p = st if _COMPUTE_MODE in (1, 3) else jnp.exp(st)
            if _COMPUTE_MODE in (2, 3):
                lsum = jnp.ones((s_loc, 1), jnp.float32)
            else:
                lsum = p.sum(axis=1, keepdims=True)
            pv = lax.dot_general(
                p.astype(jnp.bfloat16), v_t,
                (((1,), (0,)), ((), ())),
                preferred_element_type=jnp.float32,
            )
            return pv, lsum

        def make_phase(slots, mode, half=None):
            lo = 0 if half is None else half * (h2 // 2)
            n = h2 if half is None else h2 // 2

            def step(idx, c):
                bb = idx // n
                hp = lo + idx % n
                qq = q_ref[bb, hp]
                kks = [(k_ref if j is None else kbuf.at[j])[bb, hp]
                       for j in slots]
                vvs = [(v_ref if j is None else vbuf.at[j])[bb, hp]
                       for j in slots]
                kk2 = kks[0] if len(kks) == 1 else jnp.concatenate(kks, 0)
                vv2 = vvs[0] if len(vvs) == 1 else jnp.concatenate(vvs, 0)
                pvs = []
                lsums = []
                for t in range(2):
                    pv, ls = head_chunk(
                        qq[:, t * d:(t + 1) * d],
                        kk2[:, t * d:(t + 1) * d],
                        vv2[:, t * d:(t + 1) * d],
                    )
                    pvs.append(pv)
                    lsums.append(ls)
                pv = jnp.concatenate(pvs, axis=1)
                if mode == "first":
                    accbuf[bb, hp] = pv
                    for t in range(2):
                        lbuf[bb, 2 * hp + t] = lsums[t]
                elif mode == "mid":
                    accbuf[bb, hp] = accbuf[bb, hp] + pv
                    for t in range(2):
                        lbuf[bb, 2 * hp + t] = lbuf[bb, 2 * hp + t] + lsums[t]
                else:
                    tot = accbuf[bb, hp] + pv
                    outs = [
                        tot[:, t * d:(t + 1) * d]
                        / (lbuf[bb, 2 * hp + t] + lsums[t])
                        for t in range(2)
                    ]
                    out_ref[bb, hp] = jnp.concatenate(
                        outs, axis=1).astype(jnp.bfloat16)
                return c

            if not _SKIP_COMPUTE:
                lax.fori_loop(0, b * n, step, 0)

        make_phase([None], "first")

        for i in range(2):
            if not _SKIP_RING:
                k_r[i].wait_recv()
                if _do_fwd:
                    k_fwd[i].start()
                k_l[i].wait_recv()
                if _do_v:
                    v_l[i].wait_recv()
                    if _do_fwd:
                        v_fwd[i].start()
                    v_r[i].wait_recv()
            make_phase([0, 1], "mid", half=i)

        for i in range(2):
            if not _SKIP_RING and _do_fwd:
                k_fwd[i].wait_recv()
                v_fwd[i].wait_recv()
            make_phase([2], "last", half=i)

        if not _SKIP_RING:
            drain = k_r + k_l
            if _do_v:
                drain += v_r + v_l
            if _do_fwd:
                drain += k_fwd + v_fwd
            for r in drain:
                r.wait_send()

        @functools.partial(
            pl.run_scoped, second_barrier=pltpu.SemaphoreType.REGULAR
        )
        def _(second_barrier):
            for nbr in (left, right):
                pl.semaphore_signal(
                    second_barrier, inc=1,
                    device_id=(nbr,), device_id_type=pl.DeviceIdType.MESH,
                )
            pl.semaphore_wait(second_barrier, 2)

    out_p = pl.pallas_call(
        body,
        out_shape=jax.ShapeDtypeStruct((b, h2, s_loc, d2), jnp.bfloat16),
        in_specs=[
            pl.BlockSpec(memory_space=pltpu.VMEM),
            pl.BlockSpec(memory_space=pltpu.VMEM),
            pl.BlockSpec(memory_space=pltpu.VMEM),
        ],
        out_specs=pl.BlockSpec(memory_space=pltpu.VMEM),
        scratch_shapes=[
            pltpu.VMEM((3, b, h2, s_loc, d2), jnp.bfloat16),
            pltpu.VMEM((3, b, h2, s_loc, d2), jnp.bfloat16),
            pltpu.VMEM((b, h2, s_loc, d2), jnp.float32),
            pltpu.VMEM((b, h, s_loc, 1), jnp.float32),
            pltpu.SemaphoreType.DMA((12,)),
            pltpu.SemaphoreType.DMA((12,)),
        ],
        compiler_params=pltpu.CompilerParams(
            collective_id=0,
            vmem_limit_bytes=100 * 1024 * 1024,
        ),
    )(Qp, Kp, Vp)

    return out_p.transpose(0, 2, 1, 3).reshape(b, s_loc, h, d)
